# baseline (speedup 1.0000x reference)
import jax
import jax.numpy as jnp
from jax import lax
from jax.experimental import pallas as pl
from jax.experimental.pallas import tpu as pltpu


def _round_up(x, m):
    return ((x + m - 1) // m) * m


def _lstm_body(tok_ref,
               w1_ref,
               w2_ref, b2_ref,
               wd_ref, bd_ref,
               logits_ref, hn_ref, cn_ref,
               seq1_ref):
    BT, T = tok_ref.shape
    H = hn_ref.shape[1]
    Lp = w1_ref.shape[0] - H
    V = logits_ref.shape[2]
    bf16 = jnp.bfloat16
    f32 = jnp.float32
    NS = 4
    S = BT // NS

    def sig(x):
        return 0.5 * jnp.tanh(0.5 * x) + 0.5

    def act(gates, c):
        g16 = gates.astype(bf16)
        i = sig(g16[:, 0 * H:1 * H])
        f = sig(g16[:, 1 * H:2 * H])
        g = jnp.tanh(g16[:, 2 * H:3 * H])
        o = sig(g16[:, 3 * H:4 * H])
        c_new = f.astype(f32) * c + (i * g).astype(f32)
        h_new = o * jnp.tanh(c_new.astype(bf16))
        return h_new, c_new

    def act_f32(gates, c):
        i = sig(gates[:, 0 * H:1 * H])
        f = sig(gates[:, 1 * H:2 * H])
        g = jnp.tanh(gates[:, 2 * H:3 * H])
        o = sig(gates[:, 3 * H:4 * H])
        c_new = f * c + i * g
        h_new = o * jnp.tanh(c_new)
        return h_new, c_new

    lane_iota = lax.broadcasted_iota(jnp.int32, (S, Lp), 1)

    hs = [jnp.zeros((S, H), bf16)] * NS
    cs = [jnp.zeros((S, H), f32)] * NS
    for t in range(T):
        for j in range(NS):
            g = jnp.dot(
                jnp.concatenate(
                    [(lane_iota == tok_ref[j * S:(j + 1) * S, t:t + 1]).astype(bf16),
                     hs[j]], axis=1),
                w1_ref[...], preferred_element_type=f32)
            hs[j], cs[j] = act(g, cs[j])
            seq1_ref[t * BT + j * S:t * BT + (j + 1) * S, :] = hs[j]

    for t in range(T):
        r0 = t * BT
        for j in range(NS):
            gates = jnp.dot(
                jnp.concatenate([seq1_ref[r0 + j * S:r0 + (j + 1) * S, :], hs[j]],
                                axis=1),
                w2_ref[...], preferred_element_type=f32) + b2_ref[...]
            if t == T - 1:
                hf, cs[j] = act_f32(gates, cs[j])
                hn_ref[j * S:(j + 1) * S, :] = hf
                hs[j] = hf.astype(bf16)
            else:
                hs[j], cs[j] = act(gates, cs[j])
            lg = jnp.dot(hs[j], wd_ref[...],
                         preferred_element_type=f32) + bd_ref[...]
            logits_ref[j * S:(j + 1) * S, t, :] = lg[:, :V]

    for j in range(NS):
        cn_ref[j * S:(j + 1) * S, :] = cs[j]


def kernel(tokens, emb, wih1, whh1, b1, wih2, whh2, b2, wd, bd):
    B, T = tokens.shape
    V, E = emb.shape
    H = whh1.shape[0]

    BT = 512
    Bp = _round_up(B, BT)
    NB = Bp // BT
    Vp = _round_up(V, 128)
    Lp = _round_up(V, 128)

    table = jnp.pad(emb @ wih1 + b1, ((0, Lp - V), (0, 0))).astype(jnp.bfloat16)
    w1 = jnp.concatenate([table, whh1.astype(jnp.bfloat16)], axis=0)
    w2 = jnp.concatenate([wih2.astype(jnp.bfloat16),
                          whh2.astype(jnp.bfloat16)], axis=0)
    wdp = jnp.pad(wd, ((0, 0), (0, Vp - V))).astype(jnp.bfloat16)
    bdp = jnp.pad(bd, ((0, 0), (0, Vp - V)))
    toks = jnp.pad(tokens, ((0, Bp - B), (0, 0)))

    def full(shape):
        return pl.BlockSpec(shape, lambda b: (0,) * len(shape))

    logits, h_n, c_n = pl.pallas_call(
        _lstm_body,
        grid=(NB,),
        in_specs=[
            pl.BlockSpec((BT, T), lambda b: (b, 0)),
            full((Lp + H, 4 * H)), full((2 * H, 4 * H)), full((1, 4 * H)),
            full((H, Vp)), full((1, Vp)),
        ],
        out_specs=(
            pl.BlockSpec((BT, T, V), lambda b: (b, 0, 0)),
            pl.BlockSpec((BT, H), lambda b: (b, 0)),
            pl.BlockSpec((BT, H), lambda b: (b, 0)),
        ),
        out_shape=(
            jax.ShapeDtypeStruct((Bp, T, V), jnp.float32),
            jax.ShapeDtypeStruct((Bp, H), jnp.float32),
            jax.ShapeDtypeStruct((Bp, H), jnp.float32),
        ),
        scratch_shapes=[pltpu.VMEM((T * BT, H), jnp.bfloat16)],
        compiler_params=pltpu.CompilerParams(dimension_semantics=("parallel",)),
    )(toks, w1, w2, b2, wdp, bdp)

    logits = logits[:B]
    h_n = h_n[None, :B, :]
    c_n = c_n[None, :B, :]
    return logits, (h_n, c_n)


# speedup vs baseline: 1.1791x; 1.1791x over previous
import jax
import jax.numpy as jnp
from jax import lax
from jax.experimental import pallas as pl
from jax.experimental.pallas import tpu as pltpu


def _round_up(x, m):
    return ((x + m - 1) // m) * m


def _lstm_body(tok_ref,
               w1_ref,
               w2_ref, b2_ref,
               wd_ref, bd_ref,
               logits_ref, hn_ref, cn_ref,
               seq1_ref):
    BT, T = tok_ref.shape
    H = hn_ref.shape[1]
    Lp = w1_ref.shape[0] - H
    V = logits_ref.shape[2]
    bf16 = jnp.bfloat16
    f32 = jnp.float32
    NS = 4
    S = BT // NS

    def sig(x):
        return 0.5 * jnp.tanh(0.5 * x) + 0.5

    def act(gates, c):
        g16 = gates.astype(bf16)
        i = sig(g16[:, 0 * H:1 * H])
        f = sig(g16[:, 1 * H:2 * H])
        g = jnp.tanh(g16[:, 2 * H:3 * H])
        o = sig(g16[:, 3 * H:4 * H])
        c_new = f.astype(f32) * c + (i * g).astype(f32)
        h_new = o * jnp.tanh(c_new.astype(bf16))
        return h_new, c_new

    def act_f32(gates, c):
        i = sig(gates[:, 0 * H:1 * H])
        f = sig(gates[:, 1 * H:2 * H])
        g = jnp.tanh(gates[:, 2 * H:3 * H])
        o = sig(gates[:, 3 * H:4 * H])
        c_new = f * c + i * g
        h_new = o * jnp.tanh(c_new)
        return h_new, c_new

    lane_iota = lax.broadcasted_iota(jnp.int32, (S, Lp), 1)

    hs = [jnp.zeros((S, H), bf16)] * NS
    cs = [jnp.zeros((S, H), f32)] * NS
    for t in range(T):
        gs = [jnp.dot(
            jnp.concatenate(
                [(lane_iota == tok_ref[j * S:(j + 1) * S, t:t + 1]).astype(bf16),
                 hs[j]], axis=1),
            w1_ref[...], preferred_element_type=f32) for j in range(NS)]
        for j in range(NS):
            hs[j], cs[j] = act(gs[j], cs[j])
            seq1_ref[t * BT + j * S:t * BT + (j + 1) * S, :] = hs[j]

    for t in range(T):
        r0 = t * BT
        gs = [jnp.dot(
            jnp.concatenate([seq1_ref[r0 + j * S:r0 + (j + 1) * S, :], hs[j]],
                            axis=1),
            w2_ref[...], preferred_element_type=f32) + b2_ref[...]
            for j in range(NS)]
        for j in range(NS):
            gates = gs[j]
            if t == T - 1:
                hf, cs[j] = act_f32(gates, cs[j])
                hn_ref[j * S:(j + 1) * S, :] = hf
                hs[j] = hf.astype(bf16)
            else:
                hs[j], cs[j] = act(gates, cs[j])
            lg = jnp.dot(hs[j], wd_ref[...],
                         preferred_element_type=f32) + bd_ref[...]
            logits_ref[j * S:(j + 1) * S, t, :] = lg[:, :V]

    for j in range(NS):
        cn_ref[j * S:(j + 1) * S, :] = cs[j]


def kernel(tokens, emb, wih1, whh1, b1, wih2, whh2, b2, wd, bd):
    B, T = tokens.shape
    V, E = emb.shape
    H = whh1.shape[0]

    BT = 512
    Bp = _round_up(B, BT)
    NB = Bp // BT
    Vp = _round_up(V, 128)
    Lp = _round_up(V, 128)

    table = jnp.pad(emb @ wih1 + b1, ((0, Lp - V), (0, 0))).astype(jnp.bfloat16)
    w1 = jnp.concatenate([table, whh1.astype(jnp.bfloat16)], axis=0)
    w2 = jnp.concatenate([wih2.astype(jnp.bfloat16),
                          whh2.astype(jnp.bfloat16)], axis=0)
    wdp = jnp.pad(wd, ((0, 0), (0, Vp - V))).astype(jnp.bfloat16)
    bdp = jnp.pad(bd, ((0, 0), (0, Vp - V)))
    toks = jnp.pad(tokens, ((0, Bp - B), (0, 0)))

    def full(shape):
        return pl.BlockSpec(shape, lambda b: (0,) * len(shape))

    logits, h_n, c_n = pl.pallas_call(
        _lstm_body,
        grid=(NB,),
        in_specs=[
            pl.BlockSpec((BT, T), lambda b: (b, 0)),
            full((Lp + H, 4 * H)), full((2 * H, 4 * H)), full((1, 4 * H)),
            full((H, Vp)), full((1, Vp)),
        ],
        out_specs=(
            pl.BlockSpec((BT, T, V), lambda b: (b, 0, 0)),
            pl.BlockSpec((BT, H), lambda b: (b, 0)),
            pl.BlockSpec((BT, H), lambda b: (b, 0)),
        ),
        out_shape=(
            jax.ShapeDtypeStruct((Bp, T, V), jnp.float32),
            jax.ShapeDtypeStruct((Bp, H), jnp.float32),
            jax.ShapeDtypeStruct((Bp, H), jnp.float32),
        ),
        scratch_shapes=[pltpu.VMEM((T * BT, H), jnp.bfloat16)],
        compiler_params=pltpu.CompilerParams(dimension_semantics=("parallel",)),
    )(toks, w1, w2, b2, wdp, bdp)

    logits = logits[:B]
    h_n = h_n[None, :B, :]
    c_n = c_n[None, :B, :]
    return logits, (h_n, c_n)
